# Initial kernel scaffold; baseline (speedup 1.0000x reference)
import jax
import jax.numpy as jnp
from jax.experimental import pallas as pl


def kernel(user_sup_idx, user_sup_val, item_sup_idx, item_sup_val, user_inputs, item_inputs, W):
    raise NotImplementedError("write your pallas kernel here")


# speedup vs baseline: 3.2413x; 3.2413x over previous
import dataclasses
import functools

import jax
import jax.numpy as jnp
from jax import lax
from jax.experimental import pallas as pl
from jax.experimental.pallas import tpu as pltpu
from jax.experimental.pallas import tpu_sc as plsc

N = 10000
D = 128
S = 5
E = 120000

NSUB = 16
LANES = 16
WSZ = 128
EDGES = S * E
EP = ((EDGES + NSUB * WSZ - 1) // (NSUB * WSZ)) * (NSUB * WSZ)
EPW = EP // NSUB
NWIN = EPW // WSZ
NP = 10240
RCHUNK = 128
RPS = NP // NSUB


def _matmul_table(x, wt):
    m = x.shape[0]
    mb = 2000
    nb = m // mb

    def body(wt_ref, x_ref, z_ref, wacc_ref):
        i = pl.program_id(0)
        j = pl.program_id(1)

        @pl.when(jnp.logical_and(i == 0, j == 0))
        def _():
            wacc_ref[...] = jnp.zeros_like(wacc_ref)

        @pl.when(j == 0)
        def _():
            wacc_ref[...] += wt_ref[0]

        z_ref[0] = jnp.dot(x_ref[...], wacc_ref[...],
                           preferred_element_type=jnp.float32)

    return pl.pallas_call(
        body,
        grid=(S, nb),
        in_specs=[
            pl.BlockSpec((1, D, D), lambda i, j: (i, 0, 0)),
            pl.BlockSpec((mb, D), lambda i, j: (j, 0)),
        ],
        out_specs=pl.BlockSpec((1, mb, D), lambda i, j: (i, j, 0)),
        out_shape=jax.ShapeDtypeStruct((S, m, D), jnp.float32),
        scratch_shapes=[pltpu.VMEM((D, D), jnp.float32)],
    )(wt, x)


def _sc_aggregate(z, gidx, rows, vals):
    mesh = plsc.VectorSubcoreMesh(core_axis_name="c", subcore_axis_name="s")
    cp = pltpu.CompilerParams()
    if "needs_layout_passes" in pltpu.CompilerParams.__dataclass_fields__:
        cp = dataclasses.replace(cp, needs_layout_passes=False)

    @functools.partial(
        pl.kernel,
        out_type=jax.ShapeDtypeStruct((2, NP, D), jnp.float32),
        mesh=mesh,
        compiler_params=cp,
        scratch_types=[
            pltpu.VMEM((WSZ,), jnp.int32),
            pltpu.VMEM((WSZ,), jnp.int32),
            pltpu.VMEM((WSZ,), jnp.float32),
            pltpu.VMEM((WSZ, D), jnp.float32),
            pltpu.VMEM((RCHUNK, D), jnp.float32),
            pltpu.VMEM_SHARED((NP, D), jnp.float32),
        ],
    )
    def agg(z_hbm, gidx_hbm, rows_hbm, vals_hbm, out_hbm,
            idx_v, rows_v, val_v, gbuf, obuf, acc_sh):
        c = lax.axis_index("c")
        s = lax.axis_index("s")

        zero16 = jnp.zeros((LANES,), jnp.float32)

        @pl.loop(0, RCHUNK)
        def _(r):
            for k in range(D // LANES):
                obuf[r, pl.ds(k * LANES, LANES)] = zero16

        @pl.loop(0, RPS // RCHUNK)
        def _(t):
            pltpu.sync_copy(obuf, acc_sh.at[pl.ds(s * RPS + t * RCHUNK,
                                                  RCHUNK)])

        plsc.subcore_barrier()

        @pl.loop(0, NWIN)
        def _(w):
            base = c * EP + s * EPW + w * WSZ
            pltpu.sync_copy(gidx_hbm.at[pl.ds(base, WSZ)], idx_v)
            pltpu.sync_copy(rows_hbm.at[pl.ds(base, WSZ)], rows_v)
            pltpu.sync_copy(vals_hbm.at[pl.ds(base, WSZ)], val_v)
            pltpu.sync_copy(z_hbm.at[idx_v], gbuf)

            @pl.loop(0, WSZ)
            def _(e):
                vv = plsc.load_gather(val_v, [jnp.full((LANES,), e, jnp.int32)])
                for k in range(D // LANES):
                    sl = pl.ds(k * LANES, LANES)
                    gbuf[e, sl] = gbuf[e, sl] * vv

            pltpu.sync_copy(gbuf, acc_sh.at[rows_v], add=True)

        plsc.subcore_barrier()

        @pl.loop(0, RPS // RCHUNK)
        def _(t):
            r0 = s * RPS + t * RCHUNK
            pltpu.sync_copy(acc_sh.at[pl.ds(r0, RCHUNK)], obuf)

            @pl.loop(0, RCHUNK)
            def _(r):
                for k in range(D // LANES):
                    sl = pl.ds(k * LANES, LANES)
                    obuf[r, sl] = jnp.maximum(obuf[r, sl], 0.0)

            pltpu.sync_copy(obuf, out_hbm.at[c, pl.ds(r0, RCHUNK)])

    return agg(z, gidx, rows, vals)


def kernel(user_sup_idx, user_sup_val, item_sup_idx, item_sup_val,
           user_inputs, item_inputs, W):
    x = jnp.concatenate([user_inputs, item_inputs], axis=0)
    wt = jnp.transpose(W, (2, 0, 1))
    z = _matmul_table(x, wt).reshape(S * 2 * N, D)

    sup_off = (jnp.arange(S, dtype=jnp.int32) * (2 * N))[:, None]
    gidx_u = (user_sup_idx[:, 1, :] + sup_off + N).reshape(-1)
    gidx_i = (item_sup_idx[:, 1, :] + sup_off).reshape(-1)
    rows_u = user_sup_idx[:, 0, :].reshape(-1)
    rows_i = item_sup_idx[:, 0, :].reshape(-1)

    pad = EP - EDGES
    zpad_i = jnp.zeros((pad,), jnp.int32)
    zpad_f = jnp.zeros((pad,), jnp.float32)
    gidx = jnp.concatenate([gidx_u, zpad_i, gidx_i, zpad_i])
    rows = jnp.concatenate([rows_u, zpad_i, rows_i, zpad_i])
    vals = jnp.concatenate([user_sup_val.reshape(-1), zpad_f,
                            item_sup_val.reshape(-1), zpad_f])

    out = _sc_aggregate(z, gidx, rows, vals)
    return (out[0, :N], out[1, :N])


# speedup vs baseline: 5.4163x; 1.6710x over previous
import dataclasses
import functools

import jax
import jax.numpy as jnp
from jax import lax
from jax.experimental import pallas as pl
from jax.experimental.pallas import tpu as pltpu
from jax.experimental.pallas import tpu_sc as plsc

N = 10000
D = 128
S = 5
E = 120000

NSUB = 16
LANES = 16
WSZ = 128
EDGES = S * E
NWIN = 294
EPW = NWIN * WSZ
EP = NSUB * EPW
TAIL = 2 * WSZ
NP = 10240
RCHUNK = 128
RPS = NP // NSUB


def _matmul_table(x, wt):
    m = x.shape[0]
    mb = 2000
    nb = m // mb

    def body(wt_ref, x_ref, z_ref, wacc_ref):
        i = pl.program_id(0)
        j = pl.program_id(1)

        @pl.when(jnp.logical_and(i == 0, j == 0))
        def _():
            wacc_ref[...] = jnp.zeros_like(wacc_ref)

        @pl.when(j == 0)
        def _():
            wacc_ref[...] += wt_ref[0]

        z_ref[0] = jnp.dot(x_ref[...], wacc_ref[...],
                           preferred_element_type=jnp.float32)

    return pl.pallas_call(
        body,
        grid=(S, nb),
        in_specs=[
            pl.BlockSpec((1, D, D), lambda i, j: (i, 0, 0)),
            pl.BlockSpec((mb, D), lambda i, j: (j, 0)),
        ],
        out_specs=pl.BlockSpec((1, mb, D), lambda i, j: (i, j, 0)),
        out_shape=jax.ShapeDtypeStruct((S, m, D), jnp.float32),
        scratch_shapes=[pltpu.VMEM((D, D), jnp.float32)],
    )(wt, x)


def _sc_aggregate(z, gidx, rows, vals):
    mesh = plsc.VectorSubcoreMesh(core_axis_name="c", subcore_axis_name="s")
    cp = pltpu.CompilerParams()
    if "needs_layout_passes" in pltpu.CompilerParams.__dataclass_fields__:
        cp = dataclasses.replace(cp, needs_layout_passes=False)

    @functools.partial(
        pl.kernel,
        out_type=jax.ShapeDtypeStruct((2, NP, D), jnp.float32),
        mesh=mesh,
        compiler_params=cp,
        scratch_types=[
            pltpu.VMEM((WSZ,), jnp.int32),
            pltpu.VMEM((WSZ,), jnp.int32),
            pltpu.VMEM((WSZ,), jnp.int32),
            pltpu.VMEM((WSZ,), jnp.int32),
            pltpu.VMEM((WSZ,), jnp.float32),
            pltpu.VMEM((WSZ,), jnp.float32),
            pltpu.VMEM((WSZ, D), jnp.float32),
            pltpu.VMEM((WSZ, D), jnp.float32),
            pltpu.VMEM_SHARED((NP, D), jnp.float32),
            pltpu.SemaphoreType.DMA,
            pltpu.SemaphoreType.DMA,
            pltpu.SemaphoreType.DMA,
            pltpu.SemaphoreType.DMA,
        ],
    )
    def agg(z_hbm, gidx_hbm, rows_hbm, vals_hbm, out_hbm,
            idx0, idx1, rows0, rows1, val0, val1, gbuf0, gbuf1,
            acc_sh, esem0, esem1, gsem0, gsem1):
        c = lax.axis_index("c")
        s = lax.axis_index("s")
        idxb, rowsb, valb = [idx0, idx1], [rows0, rows1], [val0, val1]
        gb = [gbuf0, gbuf1]
        esem, gsem = [esem0, esem1], [gsem0, gsem1]

        def ecopies(w, b):
            base = c * EP + s * EPW + w * WSZ
            return [
                pltpu.make_async_copy(gidx_hbm.at[pl.ds(base, WSZ)], idxb[b], esem[b]),
                pltpu.make_async_copy(rows_hbm.at[pl.ds(base, WSZ)], rowsb[b], esem[b]),
                pltpu.make_async_copy(vals_hbm.at[pl.ds(base, WSZ)], valb[b], esem[b]),
            ]

        def gcopy(b):
            return pltpu.make_async_copy(z_hbm.at[idxb[b]], gb[b], gsem[b])

        for cpy in ecopies(0, 0):
            cpy.start()
        for cpy in ecopies(1, 1):
            cpy.start()

        zero16 = jnp.zeros((LANES,), jnp.float32)

        @pl.loop(0, RCHUNK)
        def _(r):
            for k in range(D // LANES):
                gbuf1[r, pl.ds(k * LANES, LANES)] = zero16

        @pl.loop(0, RPS // RCHUNK)
        def _(t):
            pltpu.sync_copy(gbuf1, acc_sh.at[pl.ds(s * RPS + t * RCHUNK,
                                                  RCHUNK)])

        for cpy in ecopies(0, 0):
            cpy.wait()
        gcopy(0).start()

        plsc.subcore_barrier()

        @pl.loop(0, NWIN, step=2)
        def _(w0):
            for p in (0, 1):
                w = w0 + p
                q = 1 - p
                gcopy(p).wait()
                for cpy in ecopies(w + 1, q):
                    cpy.wait()
                gcopy(q).start()

                @pl.loop(0, WSZ, unroll=4)
                def _(e):
                    vv = plsc.load_gather(
                        valb[p], [jnp.full((LANES,), e, jnp.int32)])
                    for k in range(D // LANES):
                        sl = pl.ds(k * LANES, LANES)
                        gb[p][e, sl] = gb[p][e, sl] * vv

                pltpu.sync_copy(gb[p], acc_sh.at[rowsb[p]], add=True)
                for cpy in ecopies(w + 2, p):
                    cpy.start()

        gcopy(0).wait()
        for cpy in ecopies(NWIN + 1, 1):
            cpy.wait()

        plsc.subcore_barrier()

        @pl.loop(0, RPS // RCHUNK)
        def _(t):
            r0 = s * RPS + t * RCHUNK
            pltpu.sync_copy(acc_sh.at[pl.ds(r0, RCHUNK)], gbuf0)

            @pl.loop(0, RCHUNK, unroll=4)
            def _(r):
                for k in range(D // LANES):
                    sl = pl.ds(k * LANES, LANES)
                    gbuf0[r, sl] = jnp.maximum(gbuf0[r, sl], 0.0)

            pltpu.sync_copy(gbuf0, out_hbm.at[c, pl.ds(r0, RCHUNK)])

    return agg(z, gidx, rows, vals)


def kernel(user_sup_idx, user_sup_val, item_sup_idx, item_sup_val,
           user_inputs, item_inputs, W):
    x = jnp.concatenate([user_inputs, item_inputs], axis=0)
    wt = jnp.transpose(W, (2, 0, 1))
    z = _matmul_table(x, wt).reshape(S * 2 * N, D)

    sup_off = (jnp.arange(S, dtype=jnp.int32) * (2 * N))[:, None]
    gidx_u = (user_sup_idx[:, 1, :] + sup_off + N).reshape(-1)
    gidx_i = (item_sup_idx[:, 1, :] + sup_off).reshape(-1)
    rows_u = user_sup_idx[:, 0, :].reshape(-1)
    rows_i = item_sup_idx[:, 0, :].reshape(-1)

    pad = EP - EDGES
    pad_g = jnp.arange(pad, dtype=jnp.int32) % (S * 2 * N)
    pad_r = jnp.arange(pad, dtype=jnp.int32) % NP
    pad_v = jnp.zeros((pad,), jnp.float32)
    tail_g = jnp.arange(TAIL, dtype=jnp.int32)
    tail_rv_i = jnp.zeros((TAIL,), jnp.int32)
    tail_rv_f = jnp.zeros((TAIL,), jnp.float32)

    gidx = jnp.concatenate([gidx_u, pad_g, gidx_i, pad_g, tail_g])
    rows = jnp.concatenate([rows_u, pad_r, rows_i, pad_r, tail_rv_i])
    vals = jnp.concatenate([user_sup_val.reshape(-1), pad_v,
                            item_sup_val.reshape(-1), pad_v, tail_rv_f])

    out = _sc_aggregate(z, gidx, rows, vals)
    return (out[0, :N], out[1, :N])


# speedup vs baseline: 6.4333x; 1.1878x over previous
import dataclasses
import functools

import jax
import jax.numpy as jnp
from jax import lax
from jax.experimental import pallas as pl
from jax.experimental.pallas import tpu as pltpu
from jax.experimental.pallas import tpu_sc as plsc

N = 10000
D = 128
S = 5
E = 120000

NSUB = 16
LANES = 16
WSZ = 128
EDGES = S * E
NWIN = 294
EPW = NWIN * WSZ
EP = NSUB * EPW
TAIL = 2 * WSZ
NP = 10240
RCHUNK = 128
RPS = NP // NSUB


def _matmul_table(x, wt):
    m = x.shape[0]
    mb = 2000
    nb = m // mb

    def body(wt_ref, x_ref, z_ref, wacc_ref):
        i = pl.program_id(0)
        j = pl.program_id(1)

        @pl.when(jnp.logical_and(i == 0, j == 0))
        def _():
            wacc_ref[...] = jnp.zeros_like(wacc_ref)

        @pl.when(j == 0)
        def _():
            wacc_ref[...] += wt_ref[0]

        z_ref[0] = jnp.dot(x_ref[...], wacc_ref[...],
                           preferred_element_type=jnp.float32)

    return pl.pallas_call(
        body,
        grid=(S, nb),
        in_specs=[
            pl.BlockSpec((1, D, D), lambda i, j: (i, 0, 0)),
            pl.BlockSpec((mb, D), lambda i, j: (j, 0)),
        ],
        out_specs=pl.BlockSpec((1, mb, D), lambda i, j: (i, j, 0)),
        out_shape=jax.ShapeDtypeStruct((S, m, D), jnp.float32),
        scratch_shapes=[pltpu.VMEM((D, D), jnp.float32)],
    )(wt, x)


def _sc_aggregate(z, gidx, rows, vals):
    mesh = plsc.VectorSubcoreMesh(core_axis_name="c", subcore_axis_name="s")
    cp = pltpu.CompilerParams()
    if "needs_layout_passes" in pltpu.CompilerParams.__dataclass_fields__:
        cp = dataclasses.replace(cp, needs_layout_passes=False)

    @functools.partial(
        pl.kernel,
        out_type=jax.ShapeDtypeStruct((2, NP, D), jnp.float32),
        mesh=mesh,
        compiler_params=cp,
        scratch_types=[
            pltpu.VMEM((WSZ,), jnp.int32),
            pltpu.VMEM((WSZ,), jnp.int32),
            pltpu.VMEM((WSZ,), jnp.int32),
            pltpu.VMEM((WSZ,), jnp.int32),
            pltpu.VMEM((WSZ,), jnp.float32),
            pltpu.VMEM((WSZ,), jnp.float32),
            pltpu.VMEM((WSZ, D), jnp.float32),
            pltpu.VMEM((WSZ, D), jnp.float32),
            pltpu.VMEM((WSZ,), jnp.int32),
            pltpu.VMEM((WSZ,), jnp.int32),
            pltpu.VMEM_SHARED((NP, D), jnp.float32),
            pltpu.SemaphoreType.DMA,
            pltpu.SemaphoreType.DMA,
            pltpu.SemaphoreType.DMA,
            pltpu.SemaphoreType.DMA,
            pltpu.SemaphoreType.DMA,
            pltpu.SemaphoreType.DMA,
        ],
    )
    def agg(z_hbm, gidx_hbm, rows_hbm, vals_hbm, out_hbm,
            idx0, idx1, rows0, rows1, val0, val1, gbuf0, gbuf1,
            srows0, srows1, acc_sh, esem0, esem1, gsem0, gsem1,
            ssem0, ssem1):
        c = lax.axis_index("c")
        s = lax.axis_index("s")
        idxb, rowsb, valb = [idx0, idx1], [rows0, rows1], [val0, val1]
        gb = [gbuf0, gbuf1]
        srows = [srows0, srows1]
        esem, gsem = [esem0, esem1], [gsem0, gsem1]
        ssem = [ssem0, ssem1]

        def ecopies(w, b):
            base = c * EP + s * EPW + w * WSZ
            return [
                pltpu.make_async_copy(gidx_hbm.at[pl.ds(base, WSZ)], idxb[b], esem[b]),
                pltpu.make_async_copy(rows_hbm.at[pl.ds(base, WSZ)], rowsb[b], esem[b]),
                pltpu.make_async_copy(vals_hbm.at[pl.ds(base, WSZ)], valb[b], esem[b]),
            ]

        def gcopy(b):
            return pltpu.make_async_copy(z_hbm.at[idxb[b]], gb[b], gsem[b])

        def scopy(b):
            return pltpu.make_async_copy(gb[b], acc_sh.at[srows[b]], ssem[b])

        for cpy in ecopies(0, 0):
            cpy.start()
        for cpy in ecopies(1, 1):
            cpy.start()

        zero16 = jnp.zeros((LANES,), jnp.float32)

        @pl.loop(0, RCHUNK)
        def _(r):
            for k in range(D // LANES):
                gbuf1[r, pl.ds(k * LANES, LANES)] = zero16

        @pl.loop(0, RPS // RCHUNK)
        def _(t):
            pltpu.sync_copy(gbuf1, acc_sh.at[pl.ds(s * RPS + t * RCHUNK,
                                                  RCHUNK)])

        for cpy in ecopies(0, 0):
            cpy.wait()
        gcopy(0).start()

        plsc.subcore_barrier()

        @pl.loop(0, NWIN, step=2)
        def _(w0):
            for p in (0, 1):
                w = w0 + p
                q = 1 - p
                gcopy(p).wait()

                @pl.when(w >= 1)
                def _():
                    scopy(q).wait()
                for cpy in ecopies(w + 1, q):
                    cpy.wait()
                gcopy(q).start()

                for k in range(WSZ // LANES):
                    sl = pl.ds(k * LANES, LANES)
                    srows[p][sl] = rowsb[p][sl]

                @pl.loop(0, WSZ, unroll=4)
                def _(e):
                    vv = plsc.load_gather(
                        valb[p], [jnp.full((LANES,), e, jnp.int32)])
                    for k in range(D // LANES):
                        sl = pl.ds(k * LANES, LANES)
                        gb[p][e, sl] = gb[p][e, sl] * vv

                pltpu.async_copy(gb[p], acc_sh.at[srows[p]], ssem[p], add=True)
                for cpy in ecopies(w + 2, p):
                    cpy.start()

        scopy(1).wait()
        gcopy(0).wait()
        for cpy in ecopies(NWIN + 1, 1):
            cpy.wait()

        plsc.subcore_barrier()

        @pl.loop(0, RPS // RCHUNK)
        def _(t):
            r0 = s * RPS + t * RCHUNK
            pltpu.sync_copy(acc_sh.at[pl.ds(r0, RCHUNK)], gbuf0)

            @pl.loop(0, RCHUNK, unroll=4)
            def _(r):
                for k in range(D // LANES):
                    sl = pl.ds(k * LANES, LANES)
                    gbuf0[r, sl] = jnp.maximum(gbuf0[r, sl], 0.0)

            pltpu.sync_copy(gbuf0, out_hbm.at[c, pl.ds(r0, RCHUNK)])

    return agg(z, gidx, rows, vals)


def kernel(user_sup_idx, user_sup_val, item_sup_idx, item_sup_val,
           user_inputs, item_inputs, W):
    x = jnp.concatenate([user_inputs, item_inputs], axis=0)
    wt = jnp.transpose(W, (2, 0, 1))
    z = _matmul_table(x, wt).reshape(S * 2 * N, D)

    sup_off = (jnp.arange(S, dtype=jnp.int32) * (2 * N))[:, None]
    gidx_u = (user_sup_idx[:, 1, :] + sup_off + N).reshape(-1)
    gidx_i = (item_sup_idx[:, 1, :] + sup_off).reshape(-1)
    rows_u = user_sup_idx[:, 0, :].reshape(-1)
    rows_i = item_sup_idx[:, 0, :].reshape(-1)

    pad = EP - EDGES
    pad_g = jnp.arange(pad, dtype=jnp.int32) % (S * 2 * N)
    pad_r = jnp.arange(pad, dtype=jnp.int32) % NP
    pad_v = jnp.zeros((pad,), jnp.float32)
    tail_g = jnp.arange(TAIL, dtype=jnp.int32)
    tail_rv_i = jnp.zeros((TAIL,), jnp.int32)
    tail_rv_f = jnp.zeros((TAIL,), jnp.float32)

    gidx = jnp.concatenate([gidx_u, pad_g, gidx_i, pad_g, tail_g])
    rows = jnp.concatenate([rows_u, pad_r, rows_i, pad_r, tail_rv_i])
    vals = jnp.concatenate([user_sup_val.reshape(-1), pad_v,
                            item_sup_val.reshape(-1), pad_v, tail_rv_f])

    out = _sc_aggregate(z, gidx, rows, vals)
    return (out[0, :N], out[1, :N])


# speedup vs baseline: 7.5722x; 1.1770x over previous
import dataclasses
import functools

import jax
import jax.numpy as jnp
from jax import lax
from jax.experimental import pallas as pl
from jax.experimental.pallas import tpu as pltpu
from jax.experimental.pallas import tpu_sc as plsc

N = 10000
D = 128
S = 5
E = 120000

NSUB = 16
LANES = 16
WSZ = 128
HALF = 64
EDGES = S * E
SLOTS = 293
LAST = SLOTS - 1
RCH = 80
NCH = N // RCH


def _matmul_table(x, wt):
    m = x.shape[0]
    mb = 2000
    nb = m // mb

    def body(wt_ref, x_ref, z_ref, wacc_ref):
        i = pl.program_id(1)

        @pl.when(i == 0)
        def _():
            wacc_ref[...] = jnp.zeros_like(wacc_ref)

        wacc_ref[...] += wt_ref[0]
        z_ref[0] = jnp.dot(x_ref[...].astype(jnp.bfloat16),
                           wacc_ref[...].astype(jnp.bfloat16),
                           preferred_element_type=jnp.float32)

    return pl.pallas_call(
        body,
        grid=(nb, S),
        in_specs=[
            pl.BlockSpec((1, D, D), lambda j, i: (i, 0, 0)),
            pl.BlockSpec((mb, D), lambda j, i: (j, 0)),
        ],
        out_specs=pl.BlockSpec((1, mb, D), lambda j, i: (i, j, 0)),
        out_shape=jax.ShapeDtypeStruct((S, m, D), jnp.float32),
        scratch_shapes=[pltpu.VMEM((D, D), jnp.float32)],
    )(wt, x)


def _sc_aggregate(z, gidx_u, rows_u, vals_u, gidx_i, rows_i, vals_i):
    mesh = plsc.VectorSubcoreMesh(core_axis_name="c", subcore_axis_name="s")
    cp = pltpu.CompilerParams()
    if "needs_layout_passes" in pltpu.CompilerParams.__dataclass_fields__:
        cp = dataclasses.replace(cp, needs_layout_passes=False)

    @functools.partial(
        pl.kernel,
        out_type=[jax.ShapeDtypeStruct((N, D), jnp.float32),
                  jax.ShapeDtypeStruct((N, D), jnp.float32)],
        mesh=mesh,
        compiler_params=cp,
        scratch_types=[
            pltpu.VMEM((WSZ,), jnp.int32),
            pltpu.VMEM((WSZ,), jnp.int32),
            pltpu.VMEM((WSZ,), jnp.int32),
            pltpu.VMEM((WSZ,), jnp.int32),
            pltpu.VMEM((WSZ,), jnp.float32),
            pltpu.VMEM((WSZ,), jnp.float32),
            pltpu.VMEM((WSZ, D), jnp.float32),
            pltpu.VMEM((WSZ, D), jnp.float32),
            pltpu.VMEM((WSZ,), jnp.int32),
            pltpu.VMEM((WSZ,), jnp.int32),
            pltpu.VMEM_SHARED((N, D), jnp.float32),
            pltpu.SemaphoreType.DMA,
            pltpu.SemaphoreType.DMA,
            pltpu.SemaphoreType.DMA,
            pltpu.SemaphoreType.DMA,
            pltpu.SemaphoreType.DMA,
            pltpu.SemaphoreType.DMA,
        ],
    )
    def agg(z_hbm, gu_hbm, ru_hbm, vu_hbm, gi_hbm, ri_hbm, vi_hbm,
            outu_hbm, outi_hbm,
            idx0, idx1, rows0, rows1, val0, val1, gbuf0, gbuf1,
            srows0, srows1, acc_sh, esem0, esem1, gsem0, gsem1,
            ssem0, ssem1):
        c = lax.axis_index("c")
        s = lax.axis_index("s")
        idxb, rowsb, valb = [idx0, idx1], [rows0, rows1], [val0, val1]
        gb = [gbuf0, gbuf1]
        srows = [srows0, srows1]
        esem, gsem = [esem0, esem1], [gsem0, gsem1]
        ssem = [ssem0, ssem1]
        zero16 = jnp.zeros((LANES,), jnp.float32)

        def run_side(e_hbm, r_hbm, v_hbm, out_hbm):
            def e_descs(w, b, sz):
                base = (s + NSUB * w) * WSZ
                dst = ((idxb[b], rowsb[b], valb[b]) if sz == WSZ else
                       (idxb[b].at[pl.ds(0, sz)], rowsb[b].at[pl.ds(0, sz)],
                        valb[b].at[pl.ds(0, sz)]))
                return [
                    pltpu.make_async_copy(e_hbm.at[pl.ds(base, sz)], dst[0], esem[b]),
                    pltpu.make_async_copy(r_hbm.at[pl.ds(base, sz)], dst[1], esem[b]),
                    pltpu.make_async_copy(v_hbm.at[pl.ds(base, sz)], dst[2], esem[b]),
                ]

            def e_start(w, b):
                part = jnp.logical_and(w == LAST, s == NSUB - 1)

                @pl.when(part)
                def _():
                    for cpy in e_descs(w, b, HALF):
                        cpy.start()

                @pl.when(jnp.logical_not(part))
                def _():
                    for cpy in e_descs(w, b, WSZ):
                        cpy.start()

            def e_wait(w, b):
                part = jnp.logical_and(w == LAST, s == NSUB - 1)

                @pl.when(part)
                def _():
                    for cpy in e_descs(w, b, HALF):
                        cpy.wait()
                    for k in range(HALF // LANES):
                        valb[b][pl.ds(HALF + k * LANES, LANES)] = zero16

                @pl.when(jnp.logical_not(part))
                def _():
                    for cpy in e_descs(w, b, WSZ):
                        cpy.wait()

            def gcopy(b):
                return pltpu.make_async_copy(z_hbm.at[idxb[b]], gb[b], gsem[b])

            def scopy(b):
                return pltpu.make_async_copy(gb[b], acc_sh.at[srows[b]], ssem[b])

            e_start(0, 0)
            e_start(1, 1)

            @pl.loop(0, RCH)
            def _(r):
                for k in range(D // LANES):
                    gbuf1[r, pl.ds(k * LANES, LANES)] = zero16

            @pl.loop(0, 8)
            def _(k):
                ch = s + NSUB * k

                @pl.when(ch < NCH)
                def _():
                    pltpu.sync_copy(gbuf1.at[pl.ds(0, RCH)],
                                    acc_sh.at[pl.ds(ch * RCH, RCH)])

            e_wait(0, 0)
            gcopy(0).start()

            plsc.subcore_barrier()

            def half_iter(w, p):
                q = 1 - p
                gcopy(p).wait()

                @pl.when(w >= 1)
                def _():
                    scopy(q).wait()

                @pl.when(w + 1 <= LAST)
                def _():
                    e_wait(w + 1, q)
                    gcopy(q).start()

                for k in range(WSZ // LANES):
                    sl = pl.ds(k * LANES, LANES)
                    srows[p][sl] = rowsb[p][sl]

                @pl.loop(0, WSZ, unroll=4)
                def _(e):
                    vv = plsc.load_gather(
                        valb[p], [jnp.full((LANES,), e, jnp.int32)])
                    for k in range(D // LANES):
                        sl = pl.ds(k * LANES, LANES)
                        gb[p][e, sl] = gb[p][e, sl] * vv

                pltpu.async_copy(gb[p], acc_sh.at[srows[p]], ssem[p], add=True)

                @pl.when(w + 2 <= LAST)
                def _():
                    e_start(w + 2, p)

            @pl.loop(0, SLOTS + 1, step=2)
            def _(w0):
                for p in (0, 1):
                    w = w0 + p

                    @pl.when(w <= LAST)
                    def _():
                        half_iter(w, p)

            scopy(LAST % 2).wait()
            plsc.subcore_barrier()

            @pl.loop(0, 8)
            def _(k):
                ch = s + NSUB * k

                @pl.when(ch < NCH)
                def _():
                    pltpu.sync_copy(acc_sh.at[pl.ds(ch * RCH, RCH)],
                                    gbuf0.at[pl.ds(0, RCH)])

                    @pl.loop(0, RCH, unroll=4)
                    def _(r):
                        for kk in range(D // LANES):
                            sl = pl.ds(kk * LANES, LANES)
                            gbuf0[r, sl] = jnp.maximum(gbuf0[r, sl], 0.0)

                    pltpu.sync_copy(gbuf0.at[pl.ds(0, RCH)],
                                    out_hbm.at[pl.ds(ch * RCH, RCH)])

        @pl.when(c == 0)
        def _():
            run_side(gu_hbm, ru_hbm, vu_hbm, outu_hbm)

        @pl.when(c == 1)
        def _():
            run_side(gi_hbm, ri_hbm, vi_hbm, outi_hbm)

    return agg(z, gidx_u, rows_u, vals_u, gidx_i, rows_i, vals_i)


def kernel(user_sup_idx, user_sup_val, item_sup_idx, item_sup_val,
           user_inputs, item_inputs, W):
    x = jnp.concatenate([user_inputs, item_inputs], axis=0)
    wt = jnp.transpose(W, (2, 0, 1))
    z = _matmul_table(x, wt).reshape(S * 2 * N, D)

    sup_off = (jnp.arange(S, dtype=jnp.int32) * (2 * N))[:, None]
    gidx_u = (user_sup_idx[:, 1, :] + sup_off + N).reshape(-1)
    gidx_i = (item_sup_idx[:, 1, :] + sup_off).reshape(-1)
    rows_u = user_sup_idx[:, 0, :].reshape(-1)
    rows_i = item_sup_idx[:, 0, :].reshape(-1)
    vals_u = user_sup_val.reshape(-1)
    vals_i = item_sup_val.reshape(-1)

    out_u, out_i = _sc_aggregate(z, gidx_u, rows_u, vals_u,
                                 gidx_i, rows_i, vals_i)
    return (out_u, out_i)
